# Initial kernel scaffold; baseline (speedup 1.0000x reference)
import functools

import jax
import jax.numpy as jnp
from jax import lax
from jax.experimental import pallas as pl
from jax.experimental.pallas import tpu as pltpu, tpu_sc as plsc

B = 1024
F = 16384
U = 4096
A = 8
L = 16


def _sc_kernel():
    info = plsc.get_sparse_core_info()
    nw = info.num_cores * info.num_subcores
    rw = B // nw
    mesh = plsc.VectorSubcoreMesh(core_axis_name="c", subcore_axis_name="s")

    @functools.partial(
        pl.kernel,
        out_type=jax.ShapeDtypeStruct((B, U), jnp.float32),
        mesh=mesh,
        scratch_types=dict(
            idx_v=pltpu.VMEM((A * U,), jnp.int32),
            w_v=pltpu.VMEM((A * U,), jnp.float32),
            bias_v=pltpu.VMEM((U,), jnp.float32),
            x0=pltpu.VMEM((F,), jnp.float32),
            x1=pltpu.VMEM((F,), jnp.float32),
            o0=pltpu.VMEM((U,), jnp.float32),
            o1=pltpu.VMEM((U,), jnp.float32),
            sx0=pltpu.SemaphoreType.DMA,
            sx1=pltpu.SemaphoreType.DMA,
            so0=pltpu.SemaphoreType.DMA,
            so1=pltpu.SemaphoreType.DMA,
        ),
    )
    def k(x_hbm, idxt_hbm, wt_hbm, bias_hbm, out_hbm, *, idx_v, w_v,
          bias_v, x0, x1, o0, o1, sx0, sx1, so0, so1):
        wid = lax.axis_index("s") * info.num_cores + lax.axis_index("c")
        base = wid * rw
        xb = [x0, x1]
        ob = [o0, o1]
        sx = [sx0, sx1]
        so = [so0, so1]

        pltpu.sync_copy(idxt_hbm, idx_v)
        pltpu.sync_copy(wt_hbm, w_v)
        pltpu.sync_copy(bias_hbm, bias_v)

        for b in range(2):
            pltpu.async_copy(x_hbm.at[base + b], xb[b], sx[b])

        @pl.loop(0, rw, step=2)
        def _rows(r0):
            for b in range(2):
                r = r0 + b
                pltpu.make_async_copy(x_hbm.at[base], xb[b], sx[b]).wait()

                @pl.when(r >= 2)
                def _():
                    pltpu.make_async_copy(ob[b], out_hbm.at[base], so[b]).wait()

                @pl.loop(0, U, step=L)
                def _cols(c):
                    acc = bias_v[pl.ds(c, L)]
                    for j in range(A):
                        iv = idx_v[pl.ds(j * U + c, L)]
                        wv = w_v[pl.ds(j * U + c, L)]
                        g = plsc.load_gather(xb[b], [iv])
                        acc = acc + wv * g
                    ob[b][pl.ds(c, L)] = acc

                @pl.when(r + 2 < rw)
                def _():
                    pltpu.async_copy(x_hbm.at[base + r + 2], xb[b], sx[b])

                pltpu.async_copy(ob[b], out_hbm.at[base + r], so[b])

        for b in range(2):
            pltpu.make_async_copy(ob[b], out_hbm.at[base], so[b]).wait()

    return k


def kernel(inputs, indices, kernel, bias):
    idx = indices.astype(jnp.int32)
    rows_b = idx[:, 0].reshape(U, A)
    cols = idx[::A, 1]
    w_b = kernel.reshape(U, A).astype(jnp.float32)
    rows_by_col = jnp.zeros((U, A), jnp.int32).at[cols].set(rows_b)
    w_by_col = jnp.zeros((U, A), jnp.float32).at[cols].set(w_b)
    idx_t = rows_by_col.T.reshape(A * U)
    w_t = w_by_col.T.reshape(A * U)
    return _sc_kernel()(inputs, idx_t, w_t, bias.astype(jnp.float32))


# speedup vs baseline: 3.2403x; 3.2403x over previous
import functools

import jax
import jax.numpy as jnp
from jax import lax
from jax.experimental import pallas as pl
from jax.experimental.pallas import tpu as pltpu, tpu_sc as plsc

B = 1024
F = 16384
U = 4096
A = 8
L = 16


def _sc_kernel():
    info = plsc.get_sparse_core_info()
    nw = info.num_cores * info.num_subcores
    rw = B // nw
    mesh = plsc.VectorSubcoreMesh(core_axis_name="c", subcore_axis_name="s")

    @functools.partial(
        pl.kernel,
        out_type=jax.ShapeDtypeStruct((B, U), jnp.float32),
        mesh=mesh,
        scratch_types=dict(
            idx_v=pltpu.VMEM((A * U,), jnp.int32),
            w_v=pltpu.VMEM((A * U,), jnp.float32),
            bias_v=pltpu.VMEM((U,), jnp.float32),
            x0=pltpu.VMEM((F,), jnp.float32),
            x1=pltpu.VMEM((F,), jnp.float32),
            o0=pltpu.VMEM((U,), jnp.float32),
            o1=pltpu.VMEM((U,), jnp.float32),
            sx0=pltpu.SemaphoreType.DMA,
            sx1=pltpu.SemaphoreType.DMA,
            so0=pltpu.SemaphoreType.DMA,
            so1=pltpu.SemaphoreType.DMA,
        ),
        compiler_params=pltpu.CompilerParams(needs_layout_passes=False),
    )
    def k(x_hbm, idxt_hbm, wt_hbm, bias_hbm, out_hbm, *, idx_v, w_v,
          bias_v, x0, x1, o0, o1, sx0, sx1, so0, so1):
        wid = lax.axis_index("s") * info.num_cores + lax.axis_index("c")
        base = wid * rw
        xb = [x0, x1]
        ob = [o0, o1]
        sx = [sx0, sx1]
        so = [so0, so1]

        pltpu.sync_copy(idxt_hbm, idx_v)
        pltpu.sync_copy(wt_hbm, w_v)
        pltpu.sync_copy(bias_hbm, bias_v)

        for b in range(2):
            pltpu.async_copy(x_hbm.at[base + b], xb[b], sx[b])

        @pl.loop(0, rw, step=2)
        def _rows(r0):
            for b in range(2):
                r = r0 + b
                pltpu.make_async_copy(x_hbm.at[base], xb[b], sx[b]).wait()

                @pl.when(r >= 2)
                def _():
                    pltpu.make_async_copy(ob[b], out_hbm.at[base], so[b]).wait()

                @pl.loop(0, U, step=L)
                def _cols(c):
                    acc = bias_v[pl.ds(c, L)]
                    for j in range(A):
                        iv = idx_v[pl.ds(j * U + c, L)]
                        wv = w_v[pl.ds(j * U + c, L)]
                        g = plsc.load_gather(xb[b], [iv])
                        acc = acc + wv * g
                    ob[b][pl.ds(c, L)] = acc

                @pl.when(r + 2 < rw)
                def _():
                    pltpu.async_copy(x_hbm.at[base + r + 2], xb[b], sx[b])

                pltpu.async_copy(ob[b], out_hbm.at[base + r], so[b])

        for b in range(2):
            pltpu.make_async_copy(ob[b], out_hbm.at[base], so[b]).wait()

    return k


def kernel(inputs, indices, kernel, bias):
    idx = indices.astype(jnp.int32)
    rows_b = idx[:, 0].reshape(U, A)
    cols = idx[::A, 1]
    w_b = kernel.reshape(U, A).astype(jnp.float32)
    rows_by_col = jnp.zeros((U, A), jnp.int32).at[cols].set(rows_b)
    w_by_col = jnp.zeros((U, A), jnp.float32).at[cols].set(w_b)
    idx_t = rows_by_col.T.reshape(A * U)
    w_t = w_by_col.T.reshape(A * U)
    return _sc_kernel()(inputs, idx_t, w_t, bias.astype(jnp.float32))


# speedup vs baseline: 3.2557x; 1.0047x over previous
import functools

import jax
import jax.numpy as jnp
from jax import lax
from jax.experimental import pallas as pl
from jax.experimental.pallas import tpu as pltpu, tpu_sc as plsc

B = 1024
F = 16384
U = 4096
A = 8
L = 16


def _sc_kernel():
    info = plsc.get_sparse_core_info()
    nw = info.num_cores * info.num_subcores
    rw = B // nw
    mesh = plsc.VectorSubcoreMesh(core_axis_name="c", subcore_axis_name="s")

    @functools.partial(
        pl.kernel,
        out_type=jax.ShapeDtypeStruct((B, U), jnp.float32),
        mesh=mesh,
        scratch_types=dict(
            idx_v=pltpu.VMEM((A * U,), jnp.int32),
            w_v=pltpu.VMEM((A * U,), jnp.float32),
            bias_v=pltpu.VMEM((U,), jnp.float32),
            x0=pltpu.VMEM((F,), jnp.float32),
            x1=pltpu.VMEM((F,), jnp.float32),
            o0=pltpu.VMEM((U,), jnp.float32),
            o1=pltpu.VMEM((U,), jnp.float32),
            sx0=pltpu.SemaphoreType.DMA,
            sx1=pltpu.SemaphoreType.DMA,
            so0=pltpu.SemaphoreType.DMA,
            so1=pltpu.SemaphoreType.DMA,
        ),
        compiler_params=pltpu.CompilerParams(needs_layout_passes=False),
    )
    def k(x_hbm, idxt_hbm, wt_hbm, bias_hbm, out_hbm, *, idx_v, w_v,
          bias_v, x0, x1, o0, o1, sx0, sx1, so0, so1):
        wid = lax.axis_index("s") * info.num_cores + lax.axis_index("c")
        base = wid * rw
        xb = [x0, x1]
        ob = [o0, o1]
        sx = [sx0, sx1]
        so = [so0, so1]

        pltpu.sync_copy(idxt_hbm, idx_v)
        pltpu.sync_copy(wt_hbm, w_v)
        pltpu.sync_copy(bias_hbm, bias_v)

        for b in range(2):
            pltpu.async_copy(x_hbm.at[base + b], xb[b], sx[b])

        @pl.loop(0, rw, step=2)
        def _rows(r0):
            for b in range(2):
                r = r0 + b
                pltpu.make_async_copy(x_hbm.at[base], xb[b], sx[b]).wait()

                @pl.when(r >= 2)
                def _():
                    pltpu.make_async_copy(ob[b], out_hbm.at[base], so[b]).wait()

                @plsc.parallel_loop(0, U, step=L, unroll=4)
                def _cols(c):
                    acc = bias_v[pl.ds(c, L)]
                    for j in range(A):
                        iv = idx_v[pl.ds(j * U + c, L)]
                        wv = w_v[pl.ds(j * U + c, L)]
                        g = plsc.load_gather(xb[b], [iv])
                        acc = acc + wv * g
                    ob[b][pl.ds(c, L)] = acc

                @pl.when(r + 2 < rw)
                def _():
                    pltpu.async_copy(x_hbm.at[base + r + 2], xb[b], sx[b])

                pltpu.async_copy(ob[b], out_hbm.at[base + r], so[b])

        for b in range(2):
            pltpu.make_async_copy(ob[b], out_hbm.at[base], so[b]).wait()

    return k


def kernel(inputs, indices, kernel, bias):
    idx = indices.astype(jnp.int32)
    rows_b = idx[:, 0].reshape(U, A)
    cols = idx[::A, 1]
    w_b = kernel.reshape(U, A).astype(jnp.float32)
    rows_by_col = jnp.zeros((U, A), jnp.int32).at[cols].set(rows_b)
    w_by_col = jnp.zeros((U, A), jnp.float32).at[cols].set(w_b)
    idx_t = rows_by_col.T.reshape(A * U)
    w_t = w_by_col.T.reshape(A * U)
    return _sc_kernel()(inputs, idx_t, w_t, bias.astype(jnp.float32))


# speedup vs baseline: 3.4248x; 1.0519x over previous
import functools

import jax
import jax.numpy as jnp
from jax import lax
from jax.experimental import pallas as pl
from jax.experimental.pallas import tpu as pltpu, tpu_sc as plsc

B = 1024
F = 16384
U = 4096
A = 8
L = 16


def _sc_kernel():
    info = plsc.get_sparse_core_info()
    nw = info.num_cores * info.num_subcores
    rw = B // nw
    npairs = rw // 2
    mesh = plsc.VectorSubcoreMesh(core_axis_name="c", subcore_axis_name="s")

    @functools.partial(
        pl.kernel,
        out_type=jax.ShapeDtypeStruct((B, U), jnp.float32),
        mesh=mesh,
        scratch_types=dict(
            idx_v=pltpu.VMEM((A * U,), jnp.int32),
            w_v=pltpu.VMEM((A * U,), jnp.float32),
            bias_v=pltpu.VMEM((U,), jnp.float32),
            x0=pltpu.VMEM((F,), jnp.float32),
            x1=pltpu.VMEM((F,), jnp.float32),
            x2=pltpu.VMEM((F,), jnp.float32),
            o_v=pltpu.VMEM((2, U), jnp.float32),
            sx0=pltpu.SemaphoreType.DMA,
            sx1=pltpu.SemaphoreType.DMA,
            sx2=pltpu.SemaphoreType.DMA,
            so=pltpu.SemaphoreType.DMA,
        ),
        compiler_params=pltpu.CompilerParams(needs_layout_passes=False),
    )
    def k(x_hbm, idxt_hbm, wt_hbm, bias_hbm, out_hbm, *, idx_v, w_v,
          bias_v, x0, x1, x2, o_v, sx0, sx1, sx2, so):
        wid = lax.axis_index("s") * info.num_cores + lax.axis_index("c")
        base = wid * rw
        xb = [x0, x1, x2]
        sx = [sx0, sx1, sx2]

        pltpu.sync_copy(idxt_hbm, idx_v)
        pltpu.sync_copy(wt_hbm, w_v)
        pltpu.sync_copy(bias_hbm, bias_v)

        for r in range(3):
            pltpu.async_copy(x_hbm.at[base + r], xb[r], sx[r])

        def do_pair(p, ba, bb):
            pltpu.make_async_copy(x_hbm.at[base], xb[ba], sx[ba]).wait()
            pltpu.make_async_copy(x_hbm.at[base], xb[bb], sx[bb]).wait()

            @pl.when(p >= 1)
            def _():
                pltpu.make_async_copy(o_v, out_hbm.at[pl.ds(base, 2)],
                                      so).wait()

            @plsc.parallel_loop(0, U, step=L, unroll=2)
            def _cols(c):
                bv = bias_v[pl.ds(c, L)]
                acc0 = bv
                acc1 = bv
                for j in range(A):
                    iv = idx_v[pl.ds(j * U + c, L)]
                    wv = w_v[pl.ds(j * U + c, L)]
                    g0 = plsc.load_gather(xb[ba], [iv])
                    g1 = plsc.load_gather(xb[bb], [iv])
                    acc0 = acc0 + wv * g0
                    acc1 = acc1 + wv * g1
                o_v[0, pl.ds(c, L)] = acc0
                o_v[1, pl.ds(c, L)] = acc1

            for d, bf in ((3, ba), (4, bb)):
                @pl.when(2 * p + d < rw)
                def _():
                    pltpu.async_copy(x_hbm.at[base + 2 * p + d], xb[bf],
                                     sx[bf])

            pltpu.async_copy(o_v, out_hbm.at[pl.ds(base + 2 * p, 2)], so)

        @pl.loop(0, npairs - 1, step=3)
        def _pairs(p0):
            for bsel in range(3):
                ba = (2 * bsel) % 3
                bb = (2 * bsel + 1) % 3
                do_pair(p0 + bsel, ba, bb)

        do_pair(npairs - 1, (2 * (npairs - 1)) % 3, (2 * (npairs - 1) + 1) % 3)
        pltpu.make_async_copy(o_v, out_hbm.at[pl.ds(base, 2)], so).wait()

    return k


def kernel(inputs, indices, kernel, bias):
    idx = indices.astype(jnp.int32)
    rows_b = idx[:, 0].reshape(U, A)
    cols = idx[::A, 1]
    w_b = kernel.reshape(U, A).astype(jnp.float32)
    rows_by_col = jnp.zeros((U, A), jnp.int32).at[cols].set(rows_b)
    w_by_col = jnp.zeros((U, A), jnp.float32).at[cols].set(w_b)
    idx_t = rows_by_col.T.reshape(A * U)
    w_t = w_by_col.T.reshape(A * U)
    return _sc_kernel()(inputs, idx_t, w_t, bias.astype(jnp.float32))


# speedup vs baseline: 4.0454x; 1.1812x over previous
import functools

import jax
import jax.numpy as jnp
from jax import lax
from jax.experimental import pallas as pl
from jax.experimental.pallas import tpu as pltpu, tpu_sc as plsc

B = 1024
F = 16384
U = 4096
A = 8
L = 16
GBLK = 4


def _sc_kernel():
    info = plsc.get_sparse_core_info()
    nw = info.num_cores * info.num_subcores
    cw = U // nw
    nsteps = cw // GBLK
    mesh = plsc.VectorSubcoreMesh(core_axis_name="c", subcore_axis_name="s")

    @functools.partial(
        pl.kernel,
        out_type=jax.ShapeDtypeStruct((U, B), jnp.float32),
        mesh=mesh,
        scratch_types=dict(
            idx_v=pltpu.VMEM((nsteps, GBLK * A), jnp.int32),
            wb_v=pltpu.VMEM((cw * A * L,), jnp.float32),
            bias_v=pltpu.VMEM((cw * L,), jnp.float32),
            g0=pltpu.VMEM((GBLK * A, B), jnp.float32),
            g1=pltpu.VMEM((GBLK * A, B), jnp.float32),
            o0=pltpu.VMEM((GBLK, B), jnp.float32),
            o1=pltpu.VMEM((GBLK, B), jnp.float32),
            sg0=pltpu.SemaphoreType.DMA,
            sg1=pltpu.SemaphoreType.DMA,
            so0=pltpu.SemaphoreType.DMA,
            so1=pltpu.SemaphoreType.DMA,
        ),
        compiler_params=pltpu.CompilerParams(needs_layout_passes=False),
    )
    def k(xt_hbm, idx_hbm, wb_hbm, bias_hbm, out_hbm, *, idx_v, wb_v,
          bias_v, g0, g1, o0, o1, sg0, sg1, so0, so1):
        wid = lax.axis_index("s") * info.num_cores + lax.axis_index("c")
        cbase = wid * cw
        gb = [g0, g1]
        ob = [o0, o1]
        sg = [sg0, sg1]
        so = [so0, so1]

        pltpu.sync_copy(idx_hbm.at[pl.ds(wid * nsteps, nsteps)], idx_v)
        pltpu.sync_copy(wb_hbm.at[pl.ds(wid * cw * A * L, cw * A * L)], wb_v)
        pltpu.sync_copy(bias_hbm.at[pl.ds(wid * cw * L, cw * L)], bias_v)

        for s in range(2):
            pltpu.async_copy(xt_hbm.at[idx_v.at[s]], gb[s], sg[s])

        @pl.loop(0, nsteps, step=2)
        def _steps(s0):
            for bsel in range(2):
                s = s0 + bsel
                pltpu.make_async_copy(xt_hbm.at[idx_v.at[s]], gb[bsel],
                                      sg[bsel]).wait()

                @pl.when(s >= 2)
                def _():
                    pltpu.make_async_copy(ob[bsel], out_hbm.at[pl.ds(0, GBLK)],
                                          so[bsel]).wait()

                for bl in range(GBLK):
                    cofs = s * GBLK + bl
                    wv = [wb_v[pl.ds((cofs * A + j) * L, L)] for j in range(A)]
                    bv = bias_v[pl.ds(cofs * L, L)]

                    @plsc.parallel_loop(0, B, step=L, unroll=2)
                    def _bt(i):
                        acc = bv
                        for j in range(A):
                            acc = acc + wv[j] * gb[bsel][bl * A + j,
                                                         pl.ds(i, L)]
                        ob[bsel][bl, pl.ds(i, L)] = acc

                @pl.when(s + 2 < nsteps)
                def _():
                    pltpu.async_copy(xt_hbm.at[idx_v.at[s + 2]], gb[bsel],
                                     sg[bsel])

                pltpu.async_copy(
                    ob[bsel], out_hbm.at[pl.ds(cbase + s * GBLK, GBLK)],
                    so[bsel])

        for bsel in range(2):
            pltpu.make_async_copy(ob[bsel], out_hbm.at[pl.ds(0, GBLK)],
                                  so[bsel]).wait()

    return k


def kernel(inputs, indices, kernel, bias):
    idx = indices.astype(jnp.int32)
    rows_b = idx[:, 0].reshape(U, A)
    cols = idx[::A, 1]
    w_b = kernel.reshape(U, A).astype(jnp.float32)
    rows_by_col = jnp.zeros((U, A), jnp.int32).at[cols].set(rows_b)
    w_by_col = jnp.zeros((U, A), jnp.float32).at[cols].set(w_b)
    idx_flat = rows_by_col.reshape(U * A).reshape(-1, GBLK * A)
    wb = jnp.broadcast_to(w_by_col.reshape(U * A, 1), (U * A, L)).reshape(-1)
    bias_b = jnp.broadcast_to(
        bias.astype(jnp.float32).reshape(U, 1), (U, L)).reshape(-1)
    xt = inputs.T
    out_t = _sc_kernel()(xt, idx_flat, wb, bias_b)
    return out_t.T
